# baseline (speedup 1.0000x reference)
import functools

import jax
import jax.numpy as jnp
from jax.experimental import pallas as pl
from jax.experimental.pallas import tpu as pltpu


def _conv_one_image(xv, w_ref, bias_row, o_ref, s_ref, m, *, C, H, W):
    HW = H * W

    rows = []
    for t in range(H // 8):
        blk = jnp.swapaxes(xv[:, 8 * t:8 * t + 8, :], 0, 1)
        rows.append(jnp.concatenate([blk[s] for s in range(8)],
                                    axis=1).astype(jnp.bfloat16))
    ctr = jnp.concatenate(rows, axis=1)

    lane = jax.lax.broadcasted_iota(jnp.int32, (C, HW), 1) % W
    zero = jnp.zeros((), jnp.bfloat16)
    zcol = jnp.zeros((C, 1), jnp.bfloat16)
    xpad = jnp.concatenate([zcol, ctr, zcol], axis=1)
    left = jnp.where(lane == 0, zero, xpad[:, 0:HW])
    right = jnp.where(lane == W - 1, zero, xpad[:, 2:HW + 2])

    zrow = jnp.zeros((3 * C, W), jnp.bfloat16)
    s_ref[m, :, :W] = zrow
    s_ref[m, :, W + HW:] = zrow
    s_ref[m, 0 * C:1 * C, W:W + HW] = left
    s_ref[m, 1 * C:2 * C, W:W + HW] = ctr
    s_ref[m, 2 * C:3 * C, W:W + HW] = right

    acc = jnp.dot(w_ref[0], s_ref[m, :, 0:HW],
                  preferred_element_type=jnp.float32)
    acc += jnp.dot(w_ref[1], s_ref[m, :, W:W + HW],
                   preferred_element_type=jnp.float32)
    acc += jnp.dot(w_ref[2], s_ref[m, :, 2 * W:2 * W + HW],
                   preferred_element_type=jnp.float32)
    acc += jnp.tile(bias_row, (1, H))

    for t in range(H // 8):
        stk = jnp.concatenate(
            [acc[None, :, (8 * t + s) * W:(8 * t + s + 1) * W]
             for s in range(8)], axis=0)
        o_ref[m, :, 8 * t:8 * t + 8, :] = jnp.swapaxes(stk, 0, 1)


def _conv3x3_kernel(x_ref, w_ref, b_ref, o_ref, s_ref, *, M, C, H, W):
    bias_row = b_ref[...]
    for m in range(M):
        _conv_one_image(x_ref[m], w_ref, bias_row, o_ref, s_ref, m,
                        C=C, H=H, W=W)


def kernel(x, weight, bias):
    N, C, H, W = x.shape
    O, _, KH, KW = weight.shape
    HW = H * W
    M = 2

    wk = jnp.transpose(weight, (2, 0, 3, 1)).reshape(
        KH, O, KW * C).astype(jnp.bfloat16)
    b2 = jnp.broadcast_to(bias.reshape(O, 1).astype(jnp.float32), (O, W))

    kfn = functools.partial(_conv3x3_kernel, M=M, C=C, H=H, W=W)
    flops = 2 * N * KH * KW * C * O * HW
    bytes_accessed = 4 * (x.size + N * O * HW) + 2 * wk.size + 4 * b2.size

    out = pl.pallas_call(
        kfn,
        out_shape=jax.ShapeDtypeStruct((N, O, H, W), jnp.float32),
        grid=(N // M,),
        in_specs=[
            pl.BlockSpec((M, C, H, W), lambda n: (n, 0, 0, 0)),
            pl.BlockSpec((KH, O, KW * C), lambda n: (0, 0, 0)),
            pl.BlockSpec((O, W), lambda n: (0, 0)),
        ],
        out_specs=pl.BlockSpec((M, O, H, W), lambda n: (n, 0, 0, 0)),
        scratch_shapes=[
            pltpu.VMEM((M, 3 * C, (H + 2) * W), jnp.bfloat16)],
        compiler_params=pltpu.CompilerParams(
            dimension_semantics=("parallel",),
            vmem_limit_bytes=56 * 1024 * 1024,
        ),
        cost_estimate=pl.CostEstimate(
            flops=flops, transcendentals=0, bytes_accessed=bytes_accessed),
    )(x, wk, b2)

    return out


# speedup vs baseline: 1.0316x; 1.0316x over previous
import functools

import jax
import jax.numpy as jnp
from jax.experimental import pallas as pl
from jax.experimental.pallas import tpu as pltpu


def _conv3x3_kernel(x_ref, w_ref, b_ref, o_ref, s_ref, *, C, H, W):
    HW = H * W
    O = o_ref.shape[1]
    xv = x_ref[0]

    for t in range(H // 8):
        blk = jnp.swapaxes(xv[:, 8 * t:8 * t + 8, :], 0, 1)
        row = jnp.concatenate([blk[s] for s in range(8)], axis=1)
        s_ref[C:2 * C, W + 8 * t * W: W + (8 * t + 8) * W] = row.astype(
            jnp.bfloat16)

    ctr = s_ref[C:2 * C, W:W + HW]

    lane = jax.lax.broadcasted_iota(jnp.int32, (C, HW), 1) % W
    zero = jnp.zeros((), jnp.bfloat16)
    zcol = jnp.zeros((C, 1), jnp.bfloat16)
    xpad = jnp.concatenate([zcol, ctr, zcol], axis=1)
    left = jnp.where(lane == 0, zero, xpad[:, 0:HW])
    right = jnp.where(lane == W - 1, zero, xpad[:, 2:HW + 2])

    zrow = jnp.zeros((3 * C, W), jnp.bfloat16)
    s_ref[:, :W] = zrow
    s_ref[:, W + HW:] = zrow
    s_ref[0 * C:1 * C, W:W + HW] = left
    s_ref[2 * C:3 * C, W:W + HW] = right

    acc = jnp.dot(w_ref[0], s_ref[:, 0:HW],
                  preferred_element_type=jnp.float32)
    acc += jnp.dot(w_ref[1], s_ref[:, W:W + HW],
                   preferred_element_type=jnp.float32)
    acc += jnp.dot(w_ref[2], s_ref[:, 2 * W:2 * W + HW],
                   preferred_element_type=jnp.float32)
    acc += jnp.tile(b_ref[...], (1, H))

    for t in range(H // 8):
        stk = jnp.concatenate(
            [acc[None, :, (8 * t + s) * W:(8 * t + s + 1) * W]
             for s in range(8)], axis=0)
        o_ref[0, :, 8 * t:8 * t + 8, :] = jnp.swapaxes(stk, 0, 1)


def kernel(x, weight, bias):
    N, C, H, W = x.shape
    O, _, KH, KW = weight.shape
    HW = H * W

    wk = jnp.transpose(weight, (2, 0, 3, 1)).reshape(
        KH, O, KW * C).astype(jnp.bfloat16)
    b2 = jnp.broadcast_to(bias.reshape(O, 1).astype(jnp.float32), (O, W))

    kfn = functools.partial(_conv3x3_kernel, C=C, H=H, W=W)
    flops = 2 * N * KH * KW * C * O * HW
    bytes_accessed = 4 * (x.size + N * O * HW) + 2 * wk.size + 4 * b2.size

    out = pl.pallas_call(
        kfn,
        out_shape=jax.ShapeDtypeStruct((N, O, H, W), jnp.float32),
        grid=(N,),
        in_specs=[
            pl.BlockSpec((1, C, H, W), lambda n: (n, 0, 0, 0)),
            pl.BlockSpec((KH, O, KW * C), lambda n: (0, 0, 0)),
            pl.BlockSpec((O, W), lambda n: (0, 0)),
        ],
        out_specs=pl.BlockSpec((1, O, H, W), lambda n: (n, 0, 0, 0)),
        scratch_shapes=[pltpu.VMEM((3 * C, (H + 2) * W), jnp.bfloat16)],
        compiler_params=pltpu.CompilerParams(
            dimension_semantics=("parallel",),
            allow_input_fusion=[False, True, True],
        ),
        cost_estimate=pl.CostEstimate(
            flops=flops, transcendentals=0, bytes_accessed=bytes_accessed),
    )(x, wk, b2)

    return out


# speedup vs baseline: 1.0407x; 1.0089x over previous
import functools

import jax
import jax.numpy as jnp
from jax.experimental import pallas as pl
from jax.experimental.pallas import tpu as pltpu


def _conv3x3_kernel(x_ref, w_ref, b_ref, o_ref, s_ref, *, C, H, W):
    HW = H * W
    O = o_ref.shape[1]
    xv = x_ref[0]

    for t in range(H // 8):
        blk = jnp.swapaxes(xv[:, 8 * t:8 * t + 8, :], 0, 1)
        row = jnp.concatenate([blk[s] for s in range(8)], axis=1)
        s_ref[C:2 * C, W + 8 * t * W: W + (8 * t + 8) * W] = row.astype(
            jnp.bfloat16)

    ctr = s_ref[C:2 * C, W:W + HW]

    lane = jax.lax.broadcasted_iota(jnp.int32, (C, HW), 1) % W
    zero = jnp.zeros((), jnp.bfloat16)
    zcol = jnp.zeros((C, 1), jnp.bfloat16)
    xpad = jnp.concatenate([zcol, ctr, zcol], axis=1)
    left = jnp.where(lane == 0, zero, xpad[:, 0:HW])
    right = jnp.where(lane == W - 1, zero, xpad[:, 2:HW + 2])

    zrow = jnp.zeros((3 * C, W), jnp.bfloat16)
    s_ref[:, :W] = zrow
    s_ref[:, W + HW:] = zrow
    s_ref[0 * C:1 * C, W:W + HW] = left
    s_ref[2 * C:3 * C, W:W + HW] = right

    acc = jnp.dot(w_ref[0], s_ref[:, 0:HW],
                  preferred_element_type=jnp.float32)
    acc += jnp.dot(w_ref[1], s_ref[:, W:W + HW],
                   preferred_element_type=jnp.float32)
    acc += jnp.dot(w_ref[2], s_ref[:, 2 * W:2 * W + HW],
                   preferred_element_type=jnp.float32)
    acc += jnp.tile(b_ref[...], (1, H))

    for t in range(H // 8):
        stk = jnp.concatenate(
            [acc[None, :, (8 * t + s) * W:(8 * t + s + 1) * W]
             for s in range(8)], axis=0)
        o_ref[0, :, 8 * t:8 * t + 8, :] = jnp.swapaxes(stk, 0, 1)


def kernel(x, weight, bias):
    N, C, H, W = x.shape
    O, _, KH, KW = weight.shape
    HW = H * W

    wk = jnp.transpose(weight, (2, 0, 3, 1)).reshape(
        KH, O, KW * C).astype(jnp.bfloat16)
    b2 = jnp.broadcast_to(bias.reshape(O, 1).astype(jnp.float32), (O, W))

    kfn = functools.partial(_conv3x3_kernel, C=C, H=H, W=W)
    flops = 2 * N * KH * KW * C * O * HW
    bytes_accessed = 4 * (x.size + N * O * HW) + 2 * wk.size + 4 * b2.size

    out = pl.pallas_call(
        kfn,
        out_shape=jax.ShapeDtypeStruct((N, O, H, W), jnp.float32),
        grid=(N,),
        in_specs=[
            pl.BlockSpec((1, C, H, W), lambda n: (n, 0, 0, 0)),
            pl.BlockSpec((KH, O, KW * C), lambda n: (0, 0, 0)),
            pl.BlockSpec((O, W), lambda n: (0, 0)),
        ],
        out_specs=pl.BlockSpec((1, O, H, W), lambda n: (n, 0, 0, 0)),
        scratch_shapes=[pltpu.VMEM((3 * C, (H + 2) * W), jnp.bfloat16)],
        compiler_params=pltpu.CompilerParams(
            dimension_semantics=("parallel",),
            allow_input_fusion=[False, True, True],
            flags={"XLA_TPU_STORE_TO_LOAD_FORWARDING_WINDOW": 12288},
        ),
        cost_estimate=pl.CostEstimate(
            flops=flops, transcendentals=0, bytes_accessed=bytes_accessed),
    )(x, wk, b2)

    return out
